# baseline (speedup 1.0000x reference)
import functools
import math

import jax
import jax.numpy as jnp
from jax import lax
from jax.experimental import pallas as pl
from jax.experimental.pallas import tpu as pltpu
from jax.experimental.pallas import tpu_sc as plsc

B = 4096
E = 64
F = 256
T = 128
LOG2T = 7
CAP = B + E * T
NT = CAP // T
NTPAD = NT + 16
NW = 32
WT = B // NW
CHW = WT // 16
G = 16
LOG2G = 4

def _zero16():
    return jnp.zeros((16,), jnp.int32)


def _ones16():
    return jnp.ones((16,), jnp.int32)


def _dyn_gather(x, idx):
    dnums = lax.GatherDimensionNumbers(
        offset_dims=(), collapsed_slice_dims=(0,), start_index_map=(0,))
    return lax.gather(x, idx[:, None], dnums, (1,),
                      mode=lax.GatherScatterMode.PROMISE_IN_BOUNDS)


def _chunk_rank(ev):
    lane = lax.iota(jnp.int32, 16)

    def body(k, r):
        vk = _dyn_gather(ev, jnp.zeros((16,), jnp.int32) + k)
        return r + ((lane > k) & (ev == vk)).astype(jnp.int32)

    return lax.fori_loop(0, 15, body, jnp.zeros((16,), jnp.int32))


def _dispatch_body(idx_hbm, x_hbm, sx_hbm, pos_hbm, te_hbm,
                   idxv, prefv, histv, basev, dv, startsv, tev, posva, posvb,
                   xv, sem, sem2, sem3):
    wid = lax.axis_index("s") * 2 + lax.axis_index("c")
    tok0 = wid * WT

    xcp = pltpu.async_copy(x_hbm.at[pl.ds(tok0, WT)], xv, sem)
    pltpu.sync_copy(idx_hbm, idxv)

    def _zero_ref(ref, nchunks):
        def zbody(i, c):
            ref[pl.ds(i * 16, 16)] = _zero16()
            return c
        lax.fori_loop(0, nchunks, zbody, 0)

    _zero_ref(prefv, 4 * E // 16)

    def _hist(ref):
        def body(c, carry):
            ev = idxv[pl.ds(c * 16, 16)]
            plsc.addupdate_scatter(ref, [ev + ((c & 3) << 6)], _ones16())
            return carry
        return body

    lax.fori_loop(0, wid * CHW, _hist(prefv), 0)

    def cpbody(i, c):
        histv[pl.ds(16 * i, 16)] = prefv[pl.ds(16 * i, 16)]
        return c
    lax.fori_loop(0, 4 * E // 16, cpbody, 0)
    lax.fori_loop(wid * CHW, B // 16, _hist(histv), 0)

    def _banksum(ref, i):
        return (ref[pl.ds(16 * i, 16)] + ref[pl.ds(E + 16 * i, 16)]
                + ref[pl.ds(2 * E + 16 * i, 16)]
                + ref[pl.ds(3 * E + 16 * i, 16)])

    def csbody(i, carry):
        tot = _banksum(histv, i)
        pt = ((tot + (T - 1)) >> LOG2T) << LOG2T
        cs = plsc.cumsum(pt)
        excl = cs - pt + carry
        basev[pl.ds(16 * i, 16)] = excl + _banksum(prefv, i)
        startsv[pl.ds(16 * i, 16)] = excl >> LOG2T
        return carry + jnp.sum(pt)
    carry = lax.fori_loop(0, E // 16, csbody, jnp.int32(0))

    @pl.when(wid == 0)
    def _():
        _zero_ref(dv, NTPAD // 16)

        def sbody(i, c):
            plsc.addupdate_scatter(dv, [startsv[pl.ds(16 * i, 16)]], _ones16())
            return c
        lax.fori_loop(0, E // 16, sbody, 0)

        def tebody(i, tc):
            dd = dv[pl.ds(16 * i, 16)]
            tev[pl.ds(16 * i, 16)] = plsc.cumsum(dd) + tc - 1
            return tc + jnp.sum(dd)
        lax.fori_loop(0, NT // 16, tebody, jnp.int32(0))
        real_tiles = carry >> LOG2T
        lrs = ((real_tiles + (G - 1)) >> LOG2G) - 1
        tev[pl.ds(NT, 16)] = jnp.zeros((16,), jnp.int32) + lrs
        pltpu.sync_copy(tev, te_hbm)

    def _poshalf(pref, h):
        def posbody(c, cc):
            ev = idxv[pl.ds((wid * CHW + h * (CHW // 2) + c) * 16, 16)]
            g = plsc.load_gather(basev, [ev])
            pref[pl.ds(c * 16, 16)] = g + _chunk_rank(ev)
            plsc.addupdate_scatter(basev, [ev], _ones16())
            return cc
        lax.fori_loop(0, CHW // 2, posbody, 0)

    HT = WT // 2
    _poshalf(posva, 0)
    xcp.wait()
    scA = pltpu.async_copy(xv.at[pl.ds(0, HT)], sx_hbm.at[posva], sem2)
    _poshalf(posvb, 1)
    scB = pltpu.async_copy(xv.at[pl.ds(HT, HT)], sx_hbm.at[posvb], sem3)
    pltpu.sync_copy(posva, pos_hbm.at[pl.ds(tok0, HT)])
    pltpu.sync_copy(posvb, pos_hbm.at[pl.ds(tok0 + HT, HT)])
    scA.wait()
    scB.wait()


def _combine_body(pos_hbm, py_hbm, out_hbm, posva, posvb, yva, yvb,
                  s1, s2, s3, s4):
    wid = lax.axis_index("s") * 2 + lax.axis_index("c")
    tok0 = wid * WT
    HT = WT // 2
    pltpu.sync_copy(pos_hbm.at[pl.ds(tok0, HT)], posva)
    gA = pltpu.async_copy(py_hbm.at[posva], yva, s1)
    pltpu.sync_copy(pos_hbm.at[pl.ds(tok0 + HT, HT)], posvb)
    gB = pltpu.async_copy(py_hbm.at[posvb], yvb, s2)
    gA.wait()
    wA = pltpu.async_copy(yva, out_hbm.at[pl.ds(tok0, HT)], s3)
    gB.wait()
    wB = pltpu.async_copy(yvb, out_hbm.at[pl.ds(tok0 + HT, HT)], s4)
    wA.wait()
    wB.wait()


def _mm_body(te_ref, xs_ref, *rest):
    ws, o_ref = rest[:-1], rest[-1]

    @pl.when(pl.program_id(0) <= te_ref[NT])
    def _():
        for g in range(G):
            sl = pl.ds(g * T, T)
            o_ref[sl] = lax.dot_general(
                xs_ref[sl], ws[g][0],
                dimension_numbers=(((1,), (1,)), ((), ())),
                preferred_element_type=jnp.float32)


def kernel(input, index, weight):
    idx = index.astype(jnp.int32)
    mesh = plsc.VectorSubcoreMesh(core_axis_name="c", subcore_axis_name="s")

    dispatch = pl.kernel(
        _dispatch_body,
        out_type=[
            jax.ShapeDtypeStruct((CAP, F), jnp.float32),
            jax.ShapeDtypeStruct((B,), jnp.int32),
            jax.ShapeDtypeStruct((NT + 16,), jnp.int32),
        ],
        mesh=mesh,
        compiler_params=pltpu.CompilerParams(needs_layout_passes=False),
        scratch_types=[
            pltpu.VMEM((B,), jnp.int32),
            pltpu.VMEM((4 * E,), jnp.int32),
            pltpu.VMEM((4 * E,), jnp.int32),
            pltpu.VMEM((E,), jnp.int32),
            pltpu.VMEM((NTPAD,), jnp.int32),
            pltpu.VMEM((E,), jnp.int32),
            pltpu.VMEM((NT + 16,), jnp.int32),
            pltpu.VMEM((WT // 2,), jnp.int32),
            pltpu.VMEM((WT // 2,), jnp.int32),
            pltpu.VMEM((WT, F), jnp.float32),
            pltpu.SemaphoreType.DMA,
            pltpu.SemaphoreType.DMA,
            pltpu.SemaphoreType.DMA,
        ],
    )
    sx, pos, te = dispatch(idx, input)

    def _w_spec(g):
        return pl.BlockSpec(
            (1, F, F),
            lambda s, te_r, g=g: (te_r[jnp.minimum(s, te_r[NT]) * G + g], 0, 0))

    def _row_spec(s, te_r):
        return (jnp.minimum(s, te_r[NT]), 0)

    grid_spec = pltpu.PrefetchScalarGridSpec(
        num_scalar_prefetch=1,
        grid=(NT // G,),
        in_specs=[pl.BlockSpec((G * T, F), _row_spec)]
                 + [_w_spec(g) for g in range(G)],
        out_specs=pl.BlockSpec((G * T, F), _row_spec),
    )
    py = pl.pallas_call(
        _mm_body, grid_spec=grid_spec,
        out_shape=jax.ShapeDtypeStruct((CAP, F), jnp.float32),
    )(te, sx, *([weight] * G))

    combine = pl.kernel(
        _combine_body,
        out_type=jax.ShapeDtypeStruct((B, F), jnp.float32),
        mesh=mesh,
        compiler_params=pltpu.CompilerParams(needs_layout_passes=False),
        scratch_types=[
            pltpu.VMEM((WT // 2,), jnp.int32),
            pltpu.VMEM((WT // 2,), jnp.int32),
            pltpu.VMEM((WT // 2, F), jnp.float32),
            pltpu.VMEM((WT // 2, F), jnp.float32),
            pltpu.SemaphoreType.DMA,
            pltpu.SemaphoreType.DMA,
            pltpu.SemaphoreType.DMA,
            pltpu.SemaphoreType.DMA,
        ],
    )
    return combine(pos, py)


# speedup vs baseline: 1.0149x; 1.0149x over previous
import functools
import math

import jax
import jax.numpy as jnp
from jax import lax
from jax.experimental import pallas as pl
from jax.experimental.pallas import tpu as pltpu
from jax.experimental.pallas import tpu_sc as plsc

B = 4096
E = 64
F = 256
T = 128
LOG2T = 7
CAP = B + E * T
NT = CAP // T
NTPAD = NT + 16
NW = 32
WT = B // NW
CHW = WT // 16
G = 16
LOG2G = 4

def _zero16():
    return jnp.zeros((16,), jnp.int32)


def _ones16():
    return jnp.ones((16,), jnp.int32)


def _dyn_gather(x, idx):
    dnums = lax.GatherDimensionNumbers(
        offset_dims=(), collapsed_slice_dims=(0,), start_index_map=(0,))
    return lax.gather(x, idx[:, None], dnums, (1,),
                      mode=lax.GatherScatterMode.PROMISE_IN_BOUNDS)


def _chunk_rank(ev):
    lane = lax.iota(jnp.int32, 16)

    def body(k, r):
        vk = _dyn_gather(ev, jnp.zeros((16,), jnp.int32) + k)
        return r + ((lane > k) & (ev == vk)).astype(jnp.int32)

    return lax.fori_loop(0, 15, body, jnp.zeros((16,), jnp.int32))


def _dispatch_body(idx_hbm, x_hbm, sx_hbm, pos_hbm, te_hbm,
                   idxv, prefv, histv, basev, dv, startsv, tev, posva, posvb,
                   xv, sem, sem2, sem3):
    wid = lax.axis_index("s") * 2 + lax.axis_index("c")
    tok0 = wid * WT

    xcp = pltpu.async_copy(x_hbm.at[pl.ds(tok0, WT)], xv, sem)
    pltpu.sync_copy(idx_hbm, idxv)

    def _zero_ref(ref, nchunks):
        def zbody(i, c):
            ref[pl.ds(i * 16, 16)] = _zero16()
            return c
        lax.fori_loop(0, nchunks, zbody, 0)

    _zero_ref(prefv, E // 16)

    def _hist(ref):
        def body(c, carry):
            ev = idxv[pl.ds(c * 16, 16)]
            plsc.addupdate_scatter(ref, [ev], _ones16())
            return carry
        return body

    lax.fori_loop(0, wid * CHW, _hist(prefv), 0)

    def cpbody(i, c):
        histv[pl.ds(16 * i, 16)] = prefv[pl.ds(16 * i, 16)]
        return c
    lax.fori_loop(0, E // 16, cpbody, 0)
    lax.fori_loop(wid * CHW, B // 16, _hist(histv), 0)

    def csbody(i, carry):
        tot = histv[pl.ds(16 * i, 16)]
        pt = ((tot + (T - 1)) >> LOG2T) << LOG2T
        cs = plsc.cumsum(pt)
        excl = cs - pt + carry
        basev[pl.ds(16 * i, 16)] = excl + prefv[pl.ds(16 * i, 16)]
        startsv[pl.ds(16 * i, 16)] = excl >> LOG2T
        return carry + jnp.sum(pt)
    carry = lax.fori_loop(0, E // 16, csbody, jnp.int32(0))

    @pl.when(wid == 0)
    def _():
        _zero_ref(dv, NTPAD // 16)

        def sbody(i, c):
            plsc.addupdate_scatter(dv, [startsv[pl.ds(16 * i, 16)]], _ones16())
            return c
        lax.fori_loop(0, E // 16, sbody, 0)

        def tebody(i, tc):
            dd = dv[pl.ds(16 * i, 16)]
            tev[pl.ds(16 * i, 16)] = plsc.cumsum(dd) + tc - 1
            return tc + jnp.sum(dd)
        lax.fori_loop(0, NT // 16, tebody, jnp.int32(0))
        real_tiles = carry >> LOG2T
        lrs = ((real_tiles + (G - 1)) >> LOG2G) - 1
        tev[pl.ds(NT, 16)] = jnp.zeros((16,), jnp.int32) + lrs
        pltpu.sync_copy(tev, te_hbm)

    def _poshalf(pref, h):
        def posbody(c, cc):
            ev = idxv[pl.ds((wid * CHW + h * (CHW // 2) + c) * 16, 16)]
            g = plsc.load_gather(basev, [ev])
            pref[pl.ds(c * 16, 16)] = g + _chunk_rank(ev)
            plsc.addupdate_scatter(basev, [ev], _ones16())
            return cc
        lax.fori_loop(0, CHW // 2, posbody, 0)

    HT = WT // 2
    _poshalf(posva, 0)
    xcp.wait()
    scA = pltpu.async_copy(xv.at[pl.ds(0, HT)], sx_hbm.at[posva], sem2)
    _poshalf(posvb, 1)
    scB = pltpu.async_copy(xv.at[pl.ds(HT, HT)], sx_hbm.at[posvb], sem3)
    pltpu.sync_copy(posva, pos_hbm.at[pl.ds(tok0, HT)])
    pltpu.sync_copy(posvb, pos_hbm.at[pl.ds(tok0 + HT, HT)])
    scA.wait()
    scB.wait()


def _combine_body(pos_hbm, py_hbm, out_hbm, posva, posvb, yva, yvb,
                  s1, s2, s3, s4):
    wid = lax.axis_index("s") * 2 + lax.axis_index("c")
    tok0 = wid * WT
    HT = WT // 2
    pltpu.sync_copy(pos_hbm.at[pl.ds(tok0, HT)], posva)
    gA = pltpu.async_copy(py_hbm.at[posva], yva, s1)
    pltpu.sync_copy(pos_hbm.at[pl.ds(tok0 + HT, HT)], posvb)
    gB = pltpu.async_copy(py_hbm.at[posvb], yvb, s2)
    gA.wait()
    wA = pltpu.async_copy(yva, out_hbm.at[pl.ds(tok0, HT)], s3)
    gB.wait()
    wB = pltpu.async_copy(yvb, out_hbm.at[pl.ds(tok0 + HT, HT)], s4)
    wA.wait()
    wB.wait()


def _mm_body(te_ref, xs_ref, *rest):
    ws, o_ref = rest[:-1], rest[-1]

    @pl.when(pl.program_id(0) <= te_ref[NT])
    def _():
        for g in range(G):
            sl = pl.ds(g * T, T)
            o_ref[sl] = lax.dot_general(
                xs_ref[sl], ws[g][0],
                dimension_numbers=(((1,), (1,)), ((), ())),
                preferred_element_type=jnp.float32)


def kernel(input, index, weight):
    idx = index.astype(jnp.int32)
    mesh = plsc.VectorSubcoreMesh(core_axis_name="c", subcore_axis_name="s")

    dispatch = pl.kernel(
        _dispatch_body,
        out_type=[
            jax.ShapeDtypeStruct((CAP, F), jnp.float32),
            jax.ShapeDtypeStruct((B,), jnp.int32),
            jax.ShapeDtypeStruct((NT + 16,), jnp.int32),
        ],
        mesh=mesh,
        compiler_params=pltpu.CompilerParams(needs_layout_passes=False),
        scratch_types=[
            pltpu.VMEM((B,), jnp.int32),
            pltpu.VMEM((E,), jnp.int32),
            pltpu.VMEM((E,), jnp.int32),
            pltpu.VMEM((E,), jnp.int32),
            pltpu.VMEM((NTPAD,), jnp.int32),
            pltpu.VMEM((E,), jnp.int32),
            pltpu.VMEM((NT + 16,), jnp.int32),
            pltpu.VMEM((WT // 2,), jnp.int32),
            pltpu.VMEM((WT // 2,), jnp.int32),
            pltpu.VMEM((WT, F), jnp.float32),
            pltpu.SemaphoreType.DMA,
            pltpu.SemaphoreType.DMA,
            pltpu.SemaphoreType.DMA,
        ],
    )
    sx, pos, te = dispatch(idx, input)

    def _w_spec(g):
        return pl.BlockSpec(
            (1, F, F),
            lambda s, te_r, g=g: (te_r[jnp.minimum(s, te_r[NT]) * G + g], 0, 0))

    def _row_spec(s, te_r):
        return (jnp.minimum(s, te_r[NT]), 0)

    grid_spec = pltpu.PrefetchScalarGridSpec(
        num_scalar_prefetch=1,
        grid=(NT // G,),
        in_specs=[pl.BlockSpec((G * T, F), _row_spec)]
                 + [_w_spec(g) for g in range(G)],
        out_specs=pl.BlockSpec((G * T, F), _row_spec),
    )
    py = pl.pallas_call(
        _mm_body, grid_spec=grid_spec,
        out_shape=jax.ShapeDtypeStruct((CAP, F), jnp.float32),
    )(te, sx, *([weight] * G))

    combine = pl.kernel(
        _combine_body,
        out_type=jax.ShapeDtypeStruct((B, F), jnp.float32),
        mesh=mesh,
        compiler_params=pltpu.CompilerParams(needs_layout_passes=False),
        scratch_types=[
            pltpu.VMEM((WT // 2,), jnp.int32),
            pltpu.VMEM((WT // 2,), jnp.int32),
            pltpu.VMEM((WT // 2, F), jnp.float32),
            pltpu.VMEM((WT // 2, F), jnp.float32),
            pltpu.SemaphoreType.DMA,
            pltpu.SemaphoreType.DMA,
            pltpu.SemaphoreType.DMA,
            pltpu.SemaphoreType.DMA,
        ],
    )
    return combine(pos, py)
